# baseline (speedup 1.0000x reference)
import functools

import jax
import jax.numpy as jnp
import numpy as np
from jax.experimental import pallas as pl
from jax.experimental.pallas import tpu as pltpu


def _phase1_kernel(x_ref, w1_ref, adj_ref, wcat_ref, hw_out_ref, t_out_ref,
                   w_store, hws, *, b, sr, h_dim):
    i = pl.program_id(0)

    @pl.when(i == 0)
    def _():
        w_store[:, :h_dim] = jnp.dot(x_ref[...], w1_ref[...],
                                     preferred_element_type=jnp.float32)
        w_store[:, h_dim:] = jnp.zeros_like(w_store[:, h_dim:])

    j = jax.lax.rem(i, sr)

    @pl.when((j == 0) & (i > 0))
    def _():
        w_store[pl.ds((i - sr) * b, sr * b), h_dim:] = hws[...]

    a = adj_ref[...]
    ct = jnp.dot(a, w_store[...], preferred_element_type=jnp.float32)
    hw_i = jnp.dot(jax.nn.relu(ct[:, :h_dim]), wcat_ref[...],
                   preferred_element_type=jnp.float32)
    hws[pl.ds(j * b, b), :] = hw_i
    hw_out_ref[...] = hw_i
    t_out_ref[...] = ct[:, h_dim:]


def _phase2_kernel(s_ref, adj_ref, hwp_ref, t_ref, noise_ref, z_ref, acc,
                   *, n, b2, bc, nbc, zdim):
    g = pl.program_id(0)
    i = s_ref[0, g]
    k = s_ref[1, g]
    first = s_ref[2, g]

    @pl.when(first == 1)
    def _():
        acc[...] = t_ref[...]

    a = adj_ref[...]
    col0 = k * bc
    ciota = jax.lax.broadcasted_iota(jnp.int32, (1, bc), 1) + col0
    a = jnp.where(ciota < n, a, 0.0)
    hwk = hwp_ref[pl.ds(k * bc, bc), :]
    riota = jax.lax.broadcasted_iota(jnp.int32, (bc, 1), 0) + col0
    hwk = jnp.where((riota >= i * b2) & (riota < n), hwk, 0.0)
    acc[...] += jnp.dot(a, hwk, preferred_element_type=jnp.float32)

    @pl.when(k == nbc - 1)
    def _():
        t = acc[...]
        mu = t[:, :zdim]
        log_sig = t[:, zdim:]
        z_ref[...] = mu + noise_ref[...] * jnp.exp(log_sig)


def _decode_kernel(zb_ref, zc_ref, out_ref):
    out_ref[...] = jax.lax.dot_general(
        zb_ref[...], zc_ref[...], (((1,), (1,)), ((), ())),
        preferred_element_type=jnp.float32)


def _p2_schedule(nsr, b2, bc, nbc):
    si, sk, sf = [], [], []
    for i in range(nsr):
        ks = (b2 * i) // bc
        for k in range(ks, nbc):
            si.append(i)
            sk.append(k)
            sf.append(1 if k == ks else 0)
    return np.array([si, sk, sf], dtype=np.int32)


def kernel(x, adj, W1, W_mu, W_sig, noise):
    n, d = x.shape
    h_dim = W1.shape[1]
    z_dim = W_mu.shape[1]
    c2 = 2 * z_dim
    b = 400 if n % 400 == 0 else n
    nb_r = n // b
    sr = 5 if nb_r % 5 == 0 else 1
    b2 = sr * b
    nsr = n // b2
    bc = 2048
    nbc = -(-n // bc)

    wcat = jnp.concatenate([W_mu, W_sig], axis=1)

    hw, t_part = pl.pallas_call(
        functools.partial(_phase1_kernel, b=b, sr=sr, h_dim=h_dim),
        grid=(nb_r,),
        in_specs=[
            pl.BlockSpec((n, d), lambda i: (0, 0)),
            pl.BlockSpec((d, h_dim), lambda i: (0, 0)),
            pl.BlockSpec((b, n), lambda i: (i, 0)),
            pl.BlockSpec((h_dim, c2), lambda i: (0, 0)),
        ],
        out_specs=[
            pl.BlockSpec((b, c2), lambda i: (i, 0)),
            pl.BlockSpec((b, c2), lambda i: (i, 0)),
        ],
        out_shape=[
            jax.ShapeDtypeStruct((nbc * bc, c2), jnp.float32),
            jax.ShapeDtypeStruct((n, c2), jnp.float32),
        ],
        scratch_shapes=[
            pltpu.VMEM((n, h_dim + c2), jnp.float32),
            pltpu.VMEM((b2, c2), jnp.float32),
        ],
    )(x, W1, adj, wcat)

    sched = jnp.asarray(_p2_schedule(nsr, b2, bc, nbc))
    g_steps = sched.shape[1]

    z = pl.pallas_call(
        functools.partial(_phase2_kernel, n=n, b2=b2, bc=bc, nbc=nbc,
                          zdim=z_dim),
        grid_spec=pltpu.PrefetchScalarGridSpec(
            num_scalar_prefetch=1,
            grid=(g_steps,),
            in_specs=[
                pl.BlockSpec((b2, bc), lambda g, s: (s[0, g], s[1, g])),
                pl.BlockSpec((nbc * bc, c2), lambda g, s: (0, 0)),
                pl.BlockSpec((b2, c2), lambda g, s: (s[0, g], 0)),
                pl.BlockSpec((b2, z_dim), lambda g, s: (s[0, g], 0)),
            ],
            out_specs=pl.BlockSpec((b2, z_dim), lambda g, s: (s[0, g], 0)),
            scratch_shapes=[pltpu.VMEM((b2, c2), jnp.float32)],
        ),
        out_shape=jax.ShapeDtypeStruct((n, z_dim), jnp.float32),
    )(sched, adj, hw, t_part, noise)

    bd = 1000 if n % 1000 == 0 else n
    bw = 5120
    nbw = -(-n // bw)
    out = pl.pallas_call(
        _decode_kernel,
        grid=(n // bd, nbw),
        in_specs=[
            pl.BlockSpec((bd, z_dim), lambda i, j: (i, 0)),
            pl.BlockSpec((bw, z_dim), lambda i, j: (j, 0)),
        ],
        out_specs=pl.BlockSpec((bd, bw), lambda i, j: (i, j)),
        out_shape=jax.ShapeDtypeStruct((n, n), jnp.float32),
    )(z, z)

    return out


# speedup vs baseline: 1.0423x; 1.0423x over previous
import functools

import jax
import jax.numpy as jnp
import numpy as np
from jax.experimental import pallas as pl
from jax.experimental.pallas import tpu as pltpu


def _phase1_kernel(x_ref, w1_ref, adj_ref, wcat_ref, hw_out_ref, t_out_ref,
                   w_store, hws, *, b, sr, h_dim):
    i = pl.program_id(0)

    @pl.when(i == 0)
    def _():
        w_store[:, :h_dim] = jnp.dot(x_ref[...], w1_ref[...],
                                     preferred_element_type=jnp.float32)
        w_store[:, h_dim:] = jnp.zeros_like(w_store[:, h_dim:])

    j = jax.lax.rem(i, sr)

    @pl.when((j == 0) & (i > 0))
    def _():
        w_store[pl.ds((i - sr) * b, sr * b), h_dim:] = hws[...]

    a = adj_ref[...]
    ct = jnp.dot(a, w_store[...], preferred_element_type=jnp.float32)
    hw_i = jnp.dot(jax.nn.relu(ct[:, :h_dim]), wcat_ref[...],
                   preferred_element_type=jnp.float32)
    hws[pl.ds(j * b, b), :] = hw_i
    hw_out_ref[...] = hw_i
    t_out_ref[...] = ct[:, h_dim:]


def _phase2_kernel(s_ref, adj_ref, hwp_ref, t_ref, noise_ref, z_ref, acc,
                   *, n, b2, bc, nbc, zdim):
    g = pl.program_id(0)
    i = s_ref[0, g]
    k = s_ref[1, g]
    first = s_ref[2, g]

    @pl.when(first == 1)
    def _():
        acc[...] = t_ref[...]

    a = adj_ref[...]
    col0 = k * bc
    ciota = jax.lax.broadcasted_iota(jnp.int32, (1, bc), 1) + col0
    a = jnp.where(ciota < n, a, 0.0)
    hwk = hwp_ref[pl.ds(k * bc, bc), :]
    riota = jax.lax.broadcasted_iota(jnp.int32, (bc, 1), 0) + col0
    hwk = jnp.where((riota >= i * b2) & (riota < n), hwk, 0.0)
    acc[...] += jnp.dot(a, hwk, preferred_element_type=jnp.float32)

    @pl.when(k == nbc - 1)
    def _():
        t = acc[...]
        mu = t[:, :zdim]
        log_sig = t[:, zdim:]
        z_ref[...] = mu + noise_ref[...] * jnp.exp(log_sig)


def _decode_kernel(zb_ref, zc_ref, out_ref):
    out_ref[...] = jax.lax.dot_general(
        zb_ref[...], zc_ref[...], (((1,), (1,)), ((), ())),
        preferred_element_type=jnp.float32)


def _p2_schedule(nsr, b2, bc, nbc):
    si, sk, sf = [], [], []
    for i in range(nsr):
        ks = (b2 * i) // bc
        for k in range(ks, nbc):
            si.append(i)
            sk.append(k)
            sf.append(1 if k == ks else 0)
    return np.array([si, sk, sf], dtype=np.int32)


def kernel(x, adj, W1, W_mu, W_sig, noise):
    n, d = x.shape
    h_dim = W1.shape[1]
    z_dim = W_mu.shape[1]
    c2 = 2 * z_dim
    b = 400 if n % 400 == 0 else n
    nb_r = n // b
    sr = 5 if nb_r % 5 == 0 else 1
    b2 = sr * b
    nsr = n // b2
    bc = 2048
    nbc = -(-n // bc)

    wcat = jnp.concatenate([W_mu, W_sig], axis=1)

    hw, t_part = pl.pallas_call(
        functools.partial(_phase1_kernel, b=b, sr=sr, h_dim=h_dim),
        grid=(nb_r,),
        in_specs=[
            pl.BlockSpec((n, d), lambda i: (0, 0)),
            pl.BlockSpec((d, h_dim), lambda i: (0, 0)),
            pl.BlockSpec((b, n), lambda i: (i, 0)),
            pl.BlockSpec((h_dim, c2), lambda i: (0, 0)),
        ],
        out_specs=[
            pl.BlockSpec((b, c2), lambda i: (i, 0)),
            pl.BlockSpec((b, c2), lambda i: (i, 0)),
        ],
        out_shape=[
            jax.ShapeDtypeStruct((nbc * bc, c2), jnp.float32),
            jax.ShapeDtypeStruct((n, c2), jnp.float32),
        ],
        scratch_shapes=[
            pltpu.VMEM((n, h_dim + c2), jnp.float32),
            pltpu.VMEM((b2, c2), jnp.float32),
        ],
    )(x, W1, adj, wcat)

    sched = jnp.asarray(_p2_schedule(nsr, b2, bc, nbc))
    g_steps = sched.shape[1]

    z = pl.pallas_call(
        functools.partial(_phase2_kernel, n=n, b2=b2, bc=bc, nbc=nbc,
                          zdim=z_dim),
        grid_spec=pltpu.PrefetchScalarGridSpec(
            num_scalar_prefetch=1,
            grid=(g_steps,),
            in_specs=[
                pl.BlockSpec((b2, bc), lambda g, s: (s[0, g], s[1, g])),
                pl.BlockSpec((nbc * bc, c2), lambda g, s: (0, 0)),
                pl.BlockSpec((b2, c2), lambda g, s: (s[0, g], 0)),
                pl.BlockSpec((b2, z_dim), lambda g, s: (s[0, g], 0)),
            ],
            out_specs=pl.BlockSpec((b2, z_dim), lambda g, s: (s[0, g], 0)),
            scratch_shapes=[pltpu.VMEM((b2, c2), jnp.float32)],
        ),
        out_shape=jax.ShapeDtypeStruct((n, z_dim), jnp.float32),
    )(sched, adj, hw, t_part, noise)

    bd = 400 if n % 400 == 0 else n
    out = pl.pallas_call(
        _decode_kernel,
        grid=(n // bd,),
        in_specs=[
            pl.BlockSpec((bd, z_dim), lambda i: (i, 0)),
            pl.BlockSpec((n, z_dim), lambda i: (0, 0)),
        ],
        out_specs=pl.BlockSpec((bd, n), lambda i: (i, 0)),
        out_shape=jax.ShapeDtypeStruct((n, n), jnp.float32),
    )(z, z)

    return out


# speedup vs baseline: 1.0597x; 1.0168x over previous
import functools

import jax
import jax.numpy as jnp
import numpy as np
from jax.experimental import pallas as pl
from jax.experimental.pallas import tpu as pltpu


def _phase1_kernel(x_ref, w1_ref, adj_ref, wcat_ref, hw_out_ref, t_out_ref,
                   w_store, hws, *, b, sr, h_dim):
    i = pl.program_id(0)

    @pl.when(i == 0)
    def _():
        w_store[:, :h_dim] = jnp.dot(x_ref[...], w1_ref[...],
                                     preferred_element_type=jnp.float32)
        w_store[:, h_dim:] = jnp.zeros_like(w_store[:, h_dim:])

    j = jax.lax.rem(i, sr)

    @pl.when((j == 0) & (i > 0))
    def _():
        w_store[pl.ds((i - sr) * b, sr * b), h_dim:] = hws[...]

    a = adj_ref[...]
    ct = jnp.dot(a, w_store[...], preferred_element_type=jnp.float32)
    hw_i = jnp.dot(jax.nn.relu(ct[:, :h_dim]), wcat_ref[...],
                   preferred_element_type=jnp.float32)
    hws[pl.ds(j * b, b), :] = hw_i
    hw_out_ref[...] = hw_i
    t_out_ref[...] = ct[:, h_dim:]


def _phase2_kernel(s_ref, adj_ref, hwp_ref, t_ref, noise_ref, z_ref, acc,
                   *, n, b2, bc, nbc, zdim):
    g = pl.program_id(0)
    i = s_ref[0, g]
    k = s_ref[1, g]
    first = s_ref[2, g]

    @pl.when(first == 1)
    def _():
        acc[...] = t_ref[...]

    a = adj_ref[...]
    col0 = k * bc
    ciota = jax.lax.broadcasted_iota(jnp.int32, (1, bc), 1) + col0
    a = jnp.where(ciota < n, a, 0.0)
    hwk = hwp_ref[pl.ds(k * bc, bc), :]
    riota = jax.lax.broadcasted_iota(jnp.int32, (bc, 1), 0) + col0
    hwk = jnp.where((riota >= i * b2) & (riota < n), hwk, 0.0)
    acc[...] += jnp.dot(a, hwk, preferred_element_type=jnp.float32)

    @pl.when(k == nbc - 1)
    def _():
        t = acc[...]
        mu = t[:, :zdim]
        log_sig = t[:, zdim:]
        z_ref[...] = mu + noise_ref[...] * jnp.exp(log_sig)


def _decode_kernel(zb_ref, zc_ref, out_ref):
    out_ref[...] = jax.lax.dot_general(
        zb_ref[...], zc_ref[...], (((1,), (1,)), ((), ())),
        preferred_element_type=jnp.float32)


def _p2_schedule(nsr, b2, bc, nbc):
    si, sk, sf = [], [], []
    for i in range(nsr):
        ks = (b2 * i) // bc
        for k in range(ks, nbc):
            si.append(i)
            sk.append(k)
            sf.append(1 if k == ks else 0)
    return np.array([si, sk, sf], dtype=np.int32)


def kernel(x, adj, W1, W_mu, W_sig, noise):
    n, d = x.shape
    h_dim = W1.shape[1]
    z_dim = W_mu.shape[1]
    c2 = 2 * z_dim
    b = 400 if n % 400 == 0 else n
    nb_r = n // b
    sr = 5 if nb_r % 5 == 0 else 1
    b2 = sr * b
    nsr = n // b2
    bc = 2560
    nbc = -(-n // bc)

    wcat = jnp.concatenate([W_mu, W_sig], axis=1)

    hw, t_part = pl.pallas_call(
        functools.partial(_phase1_kernel, b=b, sr=sr, h_dim=h_dim),
        grid=(nb_r,),
        in_specs=[
            pl.BlockSpec((n, d), lambda i: (0, 0)),
            pl.BlockSpec((d, h_dim), lambda i: (0, 0)),
            pl.BlockSpec((b, n), lambda i: (i, 0)),
            pl.BlockSpec((h_dim, c2), lambda i: (0, 0)),
        ],
        out_specs=[
            pl.BlockSpec((b, c2), lambda i: (i, 0)),
            pl.BlockSpec((b, c2), lambda i: (i, 0)),
        ],
        out_shape=[
            jax.ShapeDtypeStruct((nbc * bc, c2), jnp.float32),
            jax.ShapeDtypeStruct((n, c2), jnp.float32),
        ],
        scratch_shapes=[
            pltpu.VMEM((n, h_dim + c2), jnp.float32),
            pltpu.VMEM((b2, c2), jnp.float32),
        ],
    )(x, W1, adj, wcat)

    sched = jnp.asarray(_p2_schedule(nsr, b2, bc, nbc))
    g_steps = sched.shape[1]

    z = pl.pallas_call(
        functools.partial(_phase2_kernel, n=n, b2=b2, bc=bc, nbc=nbc,
                          zdim=z_dim),
        grid_spec=pltpu.PrefetchScalarGridSpec(
            num_scalar_prefetch=1,
            grid=(g_steps,),
            in_specs=[
                pl.BlockSpec((b2, bc), lambda g, s: (s[0, g], s[1, g])),
                pl.BlockSpec((nbc * bc, c2), lambda g, s: (0, 0)),
                pl.BlockSpec((b2, c2), lambda g, s: (s[0, g], 0)),
                pl.BlockSpec((b2, z_dim), lambda g, s: (s[0, g], 0)),
            ],
            out_specs=pl.BlockSpec((b2, z_dim), lambda g, s: (s[0, g], 0)),
            scratch_shapes=[pltpu.VMEM((b2, c2), jnp.float32)],
        ),
        out_shape=jax.ShapeDtypeStruct((n, z_dim), jnp.float32),
    )(sched, adj, hw, t_part, noise)

    bd = 400 if n % 400 == 0 else n
    out = pl.pallas_call(
        _decode_kernel,
        grid=(n // bd,),
        in_specs=[
            pl.BlockSpec((bd, z_dim), lambda i: (i, 0)),
            pl.BlockSpec((n, z_dim), lambda i: (0, 0)),
        ],
        out_specs=pl.BlockSpec((bd, n), lambda i: (i, 0)),
        out_shape=jax.ShapeDtypeStruct((n, n), jnp.float32),
    )(z, z)

    return out
